# Initial kernel scaffold; baseline (speedup 1.0000x reference)
import jax
import jax.numpy as jnp
from jax.experimental import pallas as pl


def kernel(x_note, params, edge_index_onset, edge_index_consecutive, edge_index_during, neighbor_mask_note, batch):
    raise NotImplementedError("write your pallas kernel here")


# speedup vs baseline: 1.0003x; 1.0003x over previous
import jax
import jax.numpy as jnp
from jax import lax
from jax.experimental import pallas as pl
from jax.experimental.pallas import tpu as pltpu

N = 8192; B = 8; E = 65536; LMAX = 2048
IN = 128; H = 256; ENC = 256; PC = 35; KS = 15; GH = 128


def _ln(v, w, b):
    m = v.mean(-1, keepdims=True)
    va = ((v - m) ** 2).mean(-1, keepdims=True)
    return (v - m) / jnp.sqrt(va + 1e-5) * w + b


def _gru(x, Wih, Whh, bih, bhh):
    b, l, _ = x.shape
    h0 = jnp.zeros((b, Whh.shape[1]), x.dtype)
    xW = jnp.einsum('bld,gd->blg', x, Wih) + bih
    def step(h, xw):
        gh = h @ Whh.T + bhh
        ir, iz, inn = jnp.split(xw, 3, axis=-1)
        hr, hz, hn = jnp.split(gh, 3, axis=-1)
        r = jax.nn.sigmoid(ir + hr)
        z = jax.nn.sigmoid(iz + hz)
        n = jnp.tanh(inn + r * hn)
        h2 = (1.0 - z) * n + z * h
        return h2, h2
    _, ys = jax.lax.scan(step, h0, jnp.swapaxes(xW, 0, 1))
    return jnp.swapaxes(ys, 0, 1)


def _mlp_pallas(v, W1, b1, lnw, lnb, W2, b2):
    dout = W2.shape[1]
    def body(v_ref, W1_ref, b1_ref, lnw_ref, lnb_ref, W2_ref, b2_ref, o_ref):
        u = jnp.maximum(jnp.dot(v_ref[...], W1_ref[...],
                                preferred_element_type=jnp.float32) + b1_ref[...], 0.0)
        m = u.mean(-1, keepdims=True)
        va = ((u - m) ** 2).mean(-1, keepdims=True)
        u = (u - m) / jnp.sqrt(va + 1e-5) * lnw_ref[...] + lnb_ref[...]
        o_ref[...] = jnp.dot(u, W2_ref[...], preferred_element_type=jnp.float32) + b2_ref[...]
    BLK = 1024
    grid = (N // BLK,)
    return pl.pallas_call(
        body,
        grid=grid,
        in_specs=[
            pl.BlockSpec((BLK, v.shape[1]), lambda i: (i, 0)),
            pl.BlockSpec(W1.shape, lambda i: (0, 0)),
            pl.BlockSpec((1, b1.shape[0]), lambda i: (0, 0)),
            pl.BlockSpec((1, lnw.shape[0]), lambda i: (0, 0)),
            pl.BlockSpec((1, lnb.shape[0]), lambda i: (0, 0)),
            pl.BlockSpec(W2.shape, lambda i: (0, 0)),
            pl.BlockSpec((1, b2.shape[0]), lambda i: (0, 0)),
        ],
        out_specs=pl.BlockSpec((BLK, dout), lambda i: (i, 0)),
        out_shape=jax.ShapeDtypeStruct((N, dout), jnp.float32),
    )(v, W1, b1[None], lnw[None], lnb[None], W2, b2[None])


def kernel(x_note, params, edge_index_onset, edge_index_consecutive, edge_index_during, neighbor_mask_note, batch):
    p = params
    ei_on, ei_co, ei_du = edge_index_onset, edge_index_consecutive, edge_index_during
    mask = neighbor_mask_note
    n = x_note.shape[0]

    def seg_mean(vals, idx):
        s = jax.ops.segment_sum(vals, idx, num_segments=n)
        c = jax.ops.segment_sum(jnp.ones((idx.shape[0], 1), vals.dtype), idx, num_segments=n)
        return s / jnp.clip(c, 1.0)

    def hetero(x, pre):
        out = x @ p[pre + 'root_W'] + p[pre + 'root_b']
        for name, ei in (('onset_W', ei_on), ('consecutive_W', ei_co), ('during_W', ei_du)):
            out = out + seg_mean(x[ei[0]], ei[1]) @ p[pre + name]
        return out

    h = jax.nn.relu(hetero(x_note, 'l1_'))
    h = hetero(h, 'l2_')
    lengths = jnp.bincount(batch, length=B)
    cnt = jnp.clip(lengths, 1).astype(h.dtype)[:, None]
    mean = jax.ops.segment_sum(h, batch, num_segments=B) / cnt
    hc = h - p['gn_mean_scale'] * mean[batch]
    var = jax.ops.segment_sum(hc * hc, batch, num_segments=B) / cnt
    h = hc / jnp.sqrt(var[batch] + 1e-5) * p['gn_weight'] + p['gn_bias']
    starts = jnp.concatenate([jnp.zeros((1,), lengths.dtype), jnp.cumsum(lengths)[:-1]])
    pos = jnp.arange(n) - starts[batch]
    rev = lengths[batch] - 1 - pos

    def bigru(flat, pre):
        d = flat.shape[1]
        pad_f = jnp.zeros((B, LMAX, d), flat.dtype).at[batch, pos].set(flat)
        pad_b = jnp.zeros((B, LMAX, d), flat.dtype).at[batch, rev].set(flat)
        of = _gru(pad_f, p[pre + 'f_Wih'], p[pre + 'f_Whh'], p[pre + 'f_bih'], p[pre + 'f_bhh'])
        ob = _gru(pad_b, p[pre + 'b_Wih'], p[pre + 'b_Whh'], p[pre + 'b_bih'], p[pre + 'b_bhh'])
        return jnp.concatenate([of[batch, pos], ob[batch, rev]], axis=-1)

    znote = jnp.where((mask == 0)[:, None], x_note, jnp.zeros_like(x_note))
    z = bigru(znote, 'rnn_')
    z = _ln(z, p['ln1_w'], p['ln1_b'])
    z = z @ p['proj_W'] + p['proj_b']
    x2 = jnp.concatenate([h, z], axis=-1) @ p['cat_W'] + p['cat_b']
    out_pc = _mlp_pallas(x2, p['pc1_W'], p['pc1_b'], p['pc_ln_w'], p['pc_ln_b'], p['pc2_W'], p['pc2_b'])
    x3 = jnp.concatenate([x2, out_pc], axis=-1)
    x3 = bigru(x3, 'rnnks_')
    x3 = _ln(x3, p['lnks_w'], p['lnks_b'])
    x3 = x3 @ p['projks_W'] + p['projks_b']
    out_ks = _mlp_pallas(x3, p['ks1_W'], p['ks1_b'], p['ks_ln_w'], p['ks_ln_b'], p['ks2_W'], p['ks2_b'])
    return out_pc, out_ks


# speedup vs baseline: 5.3300x; 5.3284x over previous
import jax
import jax.numpy as jnp
from jax import lax
from jax.experimental import pallas as pl
from jax.experimental.pallas import tpu as pltpu

N = 8192; B = 8; E = 65536; LMAX = 2048
IN = 128; H = 256; ENC = 256; PC = 35; KS = 15; GH = 128
CH = 128


def _bigru_kernel_body(T_ref, xwf_ref, xwb_ref, wtf_ref, wtb_ref, bhf_ref, bhb_ref,
                       of_ref, ob_ref, hf_s, hb_s):
    c = pl.program_id(0)

    @pl.when(c == 0)
    def _():
        hf_s[...] = jnp.zeros_like(hf_s)
        hb_s[...] = jnp.zeros_like(hb_s)

    @pl.when(c * CH < T_ref[0])
    def _():
        wtf = wtf_ref[...]; wtb = wtb_ref[...]
        bhf = bhf_ref[...]; bhb = bhb_ref[...]

        def gate(xw, gh, h):
            r = jax.nn.sigmoid(xw[:, :GH] + gh[:, :GH])
            z = jax.nn.sigmoid(xw[:, GH:2 * GH] + gh[:, GH:2 * GH])
            nn_ = jnp.tanh(xw[:, 2 * GH:] + r * gh[:, 2 * GH:])
            return (1.0 - z) * nn_ + z * h

        def step(t, carry):
            hf, hb = carry
            ghf = jnp.dot(hf, wtf, preferred_element_type=jnp.float32) + bhf
            ghb = jnp.dot(hb, wtb, preferred_element_type=jnp.float32) + bhb
            h2f = gate(xwf_ref[t], ghf, hf)
            h2b = gate(xwb_ref[t], ghb, hb)
            of_ref[t] = h2f
            ob_ref[t] = h2b
            return (h2f, h2b)

        hf, hb = lax.fori_loop(0, CH, step, (hf_s[...], hb_s[...]))
        hf_s[...] = hf
        hb_s[...] = hb


def _bigru_pallas(xwf, xwb, WTf, WTb, bhf, bhb, tmax):
    grid_spec = pltpu.PrefetchScalarGridSpec(
        num_scalar_prefetch=1,
        grid=(LMAX // CH,),
        in_specs=[
            pl.BlockSpec((CH, B, 3 * GH), lambda i, T: (i, 0, 0)),
            pl.BlockSpec((CH, B, 3 * GH), lambda i, T: (i, 0, 0)),
            pl.BlockSpec((GH, 3 * GH), lambda i, T: (0, 0)),
            pl.BlockSpec((GH, 3 * GH), lambda i, T: (0, 0)),
            pl.BlockSpec((1, 3 * GH), lambda i, T: (0, 0)),
            pl.BlockSpec((1, 3 * GH), lambda i, T: (0, 0)),
        ],
        out_specs=[
            pl.BlockSpec((CH, B, GH), lambda i, T: (i, 0, 0)),
            pl.BlockSpec((CH, B, GH), lambda i, T: (i, 0, 0)),
        ],
        scratch_shapes=[
            pltpu.VMEM((B, GH), jnp.float32),
            pltpu.VMEM((B, GH), jnp.float32),
        ],
    )
    return pl.pallas_call(
        _bigru_kernel_body,
        grid_spec=grid_spec,
        out_shape=[
            jax.ShapeDtypeStruct((LMAX, B, GH), jnp.float32),
            jax.ShapeDtypeStruct((LMAX, B, GH), jnp.float32),
        ],
    )(tmax, xwf, xwb, WTf, WTb, bhf, bhb)


def _ln(v, w, b):
    m = v.mean(-1, keepdims=True)
    va = ((v - m) ** 2).mean(-1, keepdims=True)
    return (v - m) / jnp.sqrt(va + 1e-5) * w + b


def _mlp_pallas(v, W1, b1, lnw, lnb, W2, b2):
    dout = W2.shape[1]
    def body(v_ref, W1_ref, b1_ref, lnw_ref, lnb_ref, W2_ref, b2_ref, o_ref):
        u = jnp.maximum(jnp.dot(v_ref[...], W1_ref[...],
                                preferred_element_type=jnp.float32) + b1_ref[...], 0.0)
        m = u.mean(-1, keepdims=True)
        va = ((u - m) ** 2).mean(-1, keepdims=True)
        u = (u - m) / jnp.sqrt(va + 1e-5) * lnw_ref[...] + lnb_ref[...]
        o_ref[...] = jnp.dot(u, W2_ref[...], preferred_element_type=jnp.float32) + b2_ref[...]
    BLK = 1024
    return pl.pallas_call(
        body,
        grid=(N // BLK,),
        in_specs=[
            pl.BlockSpec((BLK, v.shape[1]), lambda i: (i, 0)),
            pl.BlockSpec(W1.shape, lambda i: (0, 0)),
            pl.BlockSpec((1, b1.shape[0]), lambda i: (0, 0)),
            pl.BlockSpec((1, lnw.shape[0]), lambda i: (0, 0)),
            pl.BlockSpec((1, lnb.shape[0]), lambda i: (0, 0)),
            pl.BlockSpec(W2.shape, lambda i: (0, 0)),
            pl.BlockSpec((1, b2.shape[0]), lambda i: (0, 0)),
        ],
        out_specs=pl.BlockSpec((BLK, dout), lambda i: (i, 0)),
        out_shape=jax.ShapeDtypeStruct((N, dout), jnp.float32),
    )(v, W1, b1[None], lnw[None], lnb[None], W2, b2[None])


def kernel(x_note, params, edge_index_onset, edge_index_consecutive, edge_index_during, neighbor_mask_note, batch):
    p = params
    ei_on, ei_co, ei_du = edge_index_onset, edge_index_consecutive, edge_index_during
    mask = neighbor_mask_note
    n = x_note.shape[0]

    def seg_mean(vals, idx):
        s = jax.ops.segment_sum(vals, idx, num_segments=n)
        c = jax.ops.segment_sum(jnp.ones((idx.shape[0], 1), vals.dtype), idx, num_segments=n)
        return s / jnp.clip(c, 1.0)

    def hetero(x, pre):
        out = x @ p[pre + 'root_W'] + p[pre + 'root_b']
        for name, ei in (('onset_W', ei_on), ('consecutive_W', ei_co), ('during_W', ei_du)):
            out = out + seg_mean(x[ei[0]], ei[1]) @ p[pre + name]
        return out

    h = jax.nn.relu(hetero(x_note, 'l1_'))
    h = hetero(h, 'l2_')
    lengths = jnp.bincount(batch, length=B)
    cnt = jnp.clip(lengths, 1).astype(h.dtype)[:, None]
    mean = jax.ops.segment_sum(h, batch, num_segments=B) / cnt
    hc = h - p['gn_mean_scale'] * mean[batch]
    var = jax.ops.segment_sum(hc * hc, batch, num_segments=B) / cnt
    h = hc / jnp.sqrt(var[batch] + 1e-5) * p['gn_weight'] + p['gn_bias']

    starts = jnp.concatenate([jnp.zeros((1,), lengths.dtype), jnp.cumsum(lengths)[:-1]])
    pos = jnp.arange(n) - starts[batch]
    rev = lengths[batch] - 1 - pos
    tmax = jnp.max(lengths).astype(jnp.int32)[None]
    t_ar = jnp.arange(LMAX)[:, None]
    valid = t_ar < lengths[None, :]
    idx_f = jnp.where(valid, starts[None, :] + t_ar, n)
    idx_bk = jnp.where(valid, starts[None, :] + lengths[None, :] - 1 - t_ar, n)

    def bigru(flat, pre):
        xwf_flat = flat @ p[pre + 'f_Wih'].T + p[pre + 'f_bih']
        xwb_flat = flat @ p[pre + 'b_Wih'].T + p[pre + 'b_bih']
        extf = jnp.concatenate([xwf_flat, p[pre + 'f_bih'][None]], 0)
        extb = jnp.concatenate([xwb_flat, p[pre + 'b_bih'][None]], 0)
        xwf = jnp.take(extf, idx_f.reshape(-1), axis=0).reshape(LMAX, B, 3 * GH)
        xwb = jnp.take(extb, idx_bk.reshape(-1), axis=0).reshape(LMAX, B, 3 * GH)
        of, ob = _bigru_pallas(xwf, xwb, p[pre + 'f_Whh'].T, p[pre + 'b_Whh'].T,
                               p[pre + 'f_bhh'][None], p[pre + 'b_bhh'][None], tmax)
        zf = jnp.take(of.reshape(LMAX * B, GH), pos * B + batch, axis=0)
        zb = jnp.take(ob.reshape(LMAX * B, GH), rev * B + batch, axis=0)
        return jnp.concatenate([zf, zb], axis=-1)

    znote = jnp.where((mask == 0)[:, None], x_note, jnp.zeros_like(x_note))
    z = bigru(znote, 'rnn_')
    z = _ln(z, p['ln1_w'], p['ln1_b'])
    z = z @ p['proj_W'] + p['proj_b']
    x2 = jnp.concatenate([h, z], axis=-1) @ p['cat_W'] + p['cat_b']
    out_pc = _mlp_pallas(x2, p['pc1_W'], p['pc1_b'], p['pc_ln_w'], p['pc_ln_b'], p['pc2_W'], p['pc2_b'])
    x3 = jnp.concatenate([x2, out_pc], axis=-1)
    x3 = bigru(x3, 'rnnks_')
    x3 = _ln(x3, p['lnks_w'], p['lnks_b'])
    x3 = x3 @ p['projks_W'] + p['projks_b']
    out_ks = _mlp_pallas(x3, p['ks1_W'], p['ks1_b'], p['ks_ln_w'], p['ks_ln_b'], p['ks2_W'], p['ks2_b'])
    return out_pc, out_ks


# speedup vs baseline: 8.1772x; 1.5342x over previous
import functools

import jax
import jax.numpy as jnp
from jax import lax
from jax.experimental import pallas as pl
from jax.experimental.pallas import tpu as pltpu
from jax.experimental.pallas import tpu_sc as plsc

N = 8192; B = 8; E = 65536; LMAX = 2048
IN = 128; H = 256; ENC = 256; PC = 35; KS = 15; GH = 128
CH = 128

NC, NS = 2, 16
ROWS_PER_TILE = N // NS


def _sc_segsum(x_slabs, src, dst, slab2x):
    S = src.shape[0]
    K = len(x_slabs)
    EPC = E // NC
    RPT = EPC // NS // 128
    mesh = plsc.VectorSubcoreMesh(core_axis_name="c", subcore_axis_name="s")

    out_type = jax.ShapeDtypeStruct((S, NC, N, 128), jnp.float32)

    scratch = [
        pltpu.VMEM_SHARED((N, 128), jnp.float32),
        pltpu.VMEM((RPT, 128), jnp.int32),
        pltpu.VMEM((RPT, 128), jnp.int32),
        pltpu.VMEM((128, 128), jnp.float32),
        pltpu.VMEM((128, 128), jnp.float32),
        pltpu.SemaphoreType.DMA,
    ]

    def body(*refs):
        xs = refs[:K]
        src_ref, dst_ref, zeros_ref, ones_ref, out_ref = refs[K:K + 5]
        accg, idxs_v, idxd_v, rows_v, ones_v, sem = refs[K + 5:]
        c = lax.axis_index("c")
        s = lax.axis_index("s")
        my_rows = pl.ds(s * ROWS_PER_TILE, ROWS_PER_TILE)
        rbase = c * (EPC // 128) + s * RPT
        if any(t < 0 for t in slab2x):
            pltpu.sync_copy(ones_ref, ones_v)
        for t in range(S):
            pltpu.sync_copy(zeros_ref, accg.at[my_rows])
            plsc.subcore_barrier()
            pltpu.sync_copy(dst_ref.at[t, pl.ds(rbase, RPT)], idxd_v)
            if slab2x[t] >= 0:
                pltpu.sync_copy(src_ref.at[t, pl.ds(rbase, RPT)], idxs_v)
                x_ref = xs[slab2x[t]]
                for j in range(RPT):
                    pltpu.async_copy(x_ref.at[idxs_v.at[j]], rows_v, sem).wait()
                    pltpu.sync_copy(rows_v, accg.at[idxd_v.at[j]], add=True)
            else:
                for j in range(RPT):
                    pltpu.sync_copy(ones_v, accg.at[idxd_v.at[j]], add=True)
            plsc.subcore_barrier()
            pltpu.sync_copy(accg.at[my_rows], out_ref.at[t, c, my_rows])
            plsc.subcore_barrier()

    zeros = jnp.zeros((ROWS_PER_TILE, 128), jnp.float32)
    ones = jnp.ones((128, 128), jnp.float32)
    fn = pl.kernel(body, out_type=out_type, mesh=mesh, scratch_types=scratch)
    return fn(*x_slabs, src, dst, zeros, ones)


def _bigru_kernel_body(T_ref, xwf_ref, xwb_ref, wtf_ref, wtb_ref, bhf_ref, bhb_ref,
                       of_ref, ob_ref, hf_s, hb_s):
    c = pl.program_id(0)

    @pl.when(c == 0)
    def _():
        hf_s[...] = jnp.zeros_like(hf_s)
        hb_s[...] = jnp.zeros_like(hb_s)

    @pl.when(c * CH < T_ref[0])
    def _():
        wtf = wtf_ref[...]; wtb = wtb_ref[...]
        bhf = bhf_ref[...]; bhb = bhb_ref[...]

        def gate(xw, gh, h):
            r = jax.nn.sigmoid(xw[:, :GH] + gh[:, :GH])
            z = jax.nn.sigmoid(xw[:, GH:2 * GH] + gh[:, GH:2 * GH])
            nn_ = jnp.tanh(xw[:, 2 * GH:] + r * gh[:, 2 * GH:])
            return (1.0 - z) * nn_ + z * h

        def step(t, carry):
            hf, hb = carry
            ghf = jnp.dot(hf, wtf, preferred_element_type=jnp.float32) + bhf
            ghb = jnp.dot(hb, wtb, preferred_element_type=jnp.float32) + bhb
            h2f = gate(xwf_ref[t], ghf, hf)
            h2b = gate(xwb_ref[t], ghb, hb)
            of_ref[t] = h2f
            ob_ref[t] = h2b
            return (h2f, h2b)

        hf, hb = lax.fori_loop(0, CH, step, (hf_s[...], hb_s[...]))
        hf_s[...] = hf
        hb_s[...] = hb


def _bigru_pallas(xwf, xwb, WTf, WTb, bhf, bhb, tmax):
    grid_spec = pltpu.PrefetchScalarGridSpec(
        num_scalar_prefetch=1,
        grid=(LMAX // CH,),
        in_specs=[
            pl.BlockSpec((CH, B, 3 * GH), lambda i, T: (i, 0, 0)),
            pl.BlockSpec((CH, B, 3 * GH), lambda i, T: (i, 0, 0)),
            pl.BlockSpec((GH, 3 * GH), lambda i, T: (0, 0)),
            pl.BlockSpec((GH, 3 * GH), lambda i, T: (0, 0)),
            pl.BlockSpec((1, 3 * GH), lambda i, T: (0, 0)),
            pl.BlockSpec((1, 3 * GH), lambda i, T: (0, 0)),
        ],
        out_specs=[
            pl.BlockSpec((CH, B, GH), lambda i, T: (i, 0, 0)),
            pl.BlockSpec((CH, B, GH), lambda i, T: (i, 0, 0)),
        ],
        scratch_shapes=[
            pltpu.VMEM((B, GH), jnp.float32),
            pltpu.VMEM((B, GH), jnp.float32),
        ],
    )
    return pl.pallas_call(
        _bigru_kernel_body,
        grid_spec=grid_spec,
        out_shape=[
            jax.ShapeDtypeStruct((LMAX, B, GH), jnp.float32),
            jax.ShapeDtypeStruct((LMAX, B, GH), jnp.float32),
        ],
    )(tmax, xwf, xwb, WTf, WTb, bhf, bhb)


def _ln(v, w, b):
    m = v.mean(-1, keepdims=True)
    va = ((v - m) ** 2).mean(-1, keepdims=True)
    return (v - m) / jnp.sqrt(va + 1e-5) * w + b


def _mlp_pallas(v, W1, b1, lnw, lnb, W2, b2):
    dout = W2.shape[1]
    def body(v_ref, W1_ref, b1_ref, lnw_ref, lnb_ref, W2_ref, b2_ref, o_ref):
        u = jnp.maximum(jnp.dot(v_ref[...], W1_ref[...],
                                preferred_element_type=jnp.float32) + b1_ref[...], 0.0)
        m = u.mean(-1, keepdims=True)
        va = ((u - m) ** 2).mean(-1, keepdims=True)
        u = (u - m) / jnp.sqrt(va + 1e-5) * lnw_ref[...] + lnb_ref[...]
        o_ref[...] = jnp.dot(u, W2_ref[...], preferred_element_type=jnp.float32) + b2_ref[...]
    BLK = 1024
    return pl.pallas_call(
        body,
        grid=(N // BLK,),
        in_specs=[
            pl.BlockSpec((BLK, v.shape[1]), lambda i: (i, 0)),
            pl.BlockSpec(W1.shape, lambda i: (0, 0)),
            pl.BlockSpec((1, b1.shape[0]), lambda i: (0, 0)),
            pl.BlockSpec((1, lnw.shape[0]), lambda i: (0, 0)),
            pl.BlockSpec((1, lnb.shape[0]), lambda i: (0, 0)),
            pl.BlockSpec(W2.shape, lambda i: (0, 0)),
            pl.BlockSpec((1, b2.shape[0]), lambda i: (0, 0)),
        ],
        out_specs=pl.BlockSpec((BLK, dout), lambda i: (i, 0)),
        out_shape=jax.ShapeDtypeStruct((N, dout), jnp.float32),
    )(v, W1, b1[None], lnw[None], lnb[None], W2, b2[None])


def kernel(x_note, params, edge_index_onset, edge_index_consecutive, edge_index_during, neighbor_mask_note, batch):
    p = params
    ei_on, ei_co, ei_du = edge_index_onset, edge_index_consecutive, edge_index_during
    mask = neighbor_mask_note
    n = x_note.shape[0]

    src3 = jnp.stack([ei_on[0], ei_co[0], ei_du[0]]).reshape(3, E // 128, 128)
    dst3 = jnp.stack([ei_on[1], ei_co[1], ei_du[1]]).reshape(3, E // 128, 128)
    out1 = _sc_segsum([x_note], jnp.concatenate([src3, src3], 0),
                      jnp.concatenate([dst3, dst3], 0),
                      (0, 0, 0, -1, -1, -1))
    sums1 = out1[:3]
    cnt3 = jnp.clip(out1[3:, 0, :, 0] + out1[3:, 1, :, 0], 1.0)[..., None]
    mean1 = (sums1[:, 0] + sums1[:, 1]) / cnt3
    h = x_note @ p['l1_root_W'] + p['l1_root_b']
    for t, name in enumerate(('onset_W', 'consecutive_W', 'during_W')):
        h = h + mean1[t] @ p['l1_' + name]
    h = jax.nn.relu(h)

    src6 = jnp.stack([src3[0], src3[0], src3[1], src3[1], src3[2], src3[2]])
    dst6 = jnp.stack([dst3[0], dst3[0], dst3[1], dst3[1], dst3[2], dst3[2]])
    sums2 = _sc_segsum([h[:, :128], h[:, 128:]], src6, dst6,
                       (0, 1, 0, 1, 0, 1))
    sums2 = sums2[:, 0] + sums2[:, 1]
    h2 = h @ p['l2_root_W'] + p['l2_root_b']
    for t, name in enumerate(('onset_W', 'consecutive_W', 'during_W')):
        mean2 = jnp.concatenate([sums2[2 * t], sums2[2 * t + 1]], -1) / cnt3[t]
        h2 = h2 + mean2 @ p['l2_' + name]
    h = h2
    lengths = jnp.bincount(batch, length=B)
    cnt = jnp.clip(lengths, 1).astype(h.dtype)[:, None]
    mean = jax.ops.segment_sum(h, batch, num_segments=B) / cnt
    hc = h - p['gn_mean_scale'] * mean[batch]
    var = jax.ops.segment_sum(hc * hc, batch, num_segments=B) / cnt
    h = hc / jnp.sqrt(var[batch] + 1e-5) * p['gn_weight'] + p['gn_bias']

    starts = jnp.concatenate([jnp.zeros((1,), lengths.dtype), jnp.cumsum(lengths)[:-1]])
    pos = jnp.arange(n) - starts[batch]
    rev = lengths[batch] - 1 - pos
    tmax = jnp.max(lengths).astype(jnp.int32)[None]
    t_ar = jnp.arange(LMAX)[:, None]
    valid = t_ar < lengths[None, :]
    idx_f = jnp.where(valid, starts[None, :] + t_ar, n)
    idx_bk = jnp.where(valid, starts[None, :] + lengths[None, :] - 1 - t_ar, n)

    def bigru(flat, pre):
        xwf_flat = flat @ p[pre + 'f_Wih'].T + p[pre + 'f_bih']
        xwb_flat = flat @ p[pre + 'b_Wih'].T + p[pre + 'b_bih']
        extf = jnp.concatenate([xwf_flat, p[pre + 'f_bih'][None]], 0)
        extb = jnp.concatenate([xwb_flat, p[pre + 'b_bih'][None]], 0)
        xwf = jnp.take(extf, idx_f.reshape(-1), axis=0).reshape(LMAX, B, 3 * GH)
        xwb = jnp.take(extb, idx_bk.reshape(-1), axis=0).reshape(LMAX, B, 3 * GH)
        of, ob = _bigru_pallas(xwf, xwb, p[pre + 'f_Whh'].T, p[pre + 'b_Whh'].T,
                               p[pre + 'f_bhh'][None], p[pre + 'b_bhh'][None], tmax)
        zf = jnp.take(of.reshape(LMAX * B, GH), pos * B + batch, axis=0)
        zb = jnp.take(ob.reshape(LMAX * B, GH), rev * B + batch, axis=0)
        return jnp.concatenate([zf, zb], axis=-1)

    znote = jnp.where((mask == 0)[:, None], x_note, jnp.zeros_like(x_note))
    z = bigru(znote, 'rnn_')
    z = _ln(z, p['ln1_w'], p['ln1_b'])
    z = z @ p['proj_W'] + p['proj_b']
    x2 = jnp.concatenate([h, z], axis=-1) @ p['cat_W'] + p['cat_b']
    out_pc = _mlp_pallas(x2, p['pc1_W'], p['pc1_b'], p['pc_ln_w'], p['pc_ln_b'], p['pc2_W'], p['pc2_b'])
    x3 = jnp.concatenate([x2, out_pc], axis=-1)
    x3 = bigru(x3, 'rnnks_')
    x3 = _ln(x3, p['lnks_w'], p['lnks_b'])
    x3 = x3 @ p['projks_W'] + p['projks_b']
    out_ks = _mlp_pallas(x3, p['ks1_W'], p['ks1_b'], p['ks_ln_w'], p['ks_ln_b'], p['ks2_W'], p['ks2_b'])
    return out_pc, out_ks


# speedup vs baseline: 8.4219x; 1.0299x over previous
import functools

import jax
import jax.numpy as jnp
from jax import lax
from jax.experimental import pallas as pl
from jax.experimental.pallas import tpu as pltpu
from jax.experimental.pallas import tpu_sc as plsc

N = 8192; B = 8; E = 65536; LMAX = 2048
IN = 128; H = 256; ENC = 256; PC = 35; KS = 15; GH = 128
CH = 128

NC, NS = 2, 16
ROWS_PER_TILE = N // NS


def _sc_segsum(x_slabs, src, dst, slab2x):
    S = src.shape[0]
    K = len(x_slabs)
    EPC = E // NC
    RPT = EPC // NS // 128
    mesh = plsc.VectorSubcoreMesh(core_axis_name="c", subcore_axis_name="s")

    out_type = jax.ShapeDtypeStruct((S, NC, N, 128), jnp.float32)

    scratch = [
        pltpu.VMEM_SHARED((N, 128), jnp.float32),
        pltpu.VMEM((RPT, 128), jnp.int32),
        pltpu.VMEM((RPT, 128), jnp.int32),
        pltpu.VMEM((128, 128), jnp.float32),
        pltpu.VMEM((128, 128), jnp.float32),
        pltpu.SemaphoreType.DMA,
    ]

    def body(*refs):
        xs = refs[:K]
        src_ref, dst_ref, zeros_ref, ones_ref, out_ref = refs[K:K + 5]
        accg, idxs_v, idxd_v, rows_v, ones_v, sem = refs[K + 5:]
        c = lax.axis_index("c")
        s = lax.axis_index("s")
        my_rows = pl.ds(s * ROWS_PER_TILE, ROWS_PER_TILE)
        rbase = c * (EPC // 128) + s * RPT
        if any(t < 0 for t in slab2x):
            pltpu.sync_copy(ones_ref, ones_v)
        for t in range(S):
            pltpu.sync_copy(zeros_ref, accg.at[my_rows])
            plsc.subcore_barrier()
            pltpu.sync_copy(dst_ref.at[t, pl.ds(rbase, RPT)], idxd_v)
            if slab2x[t] >= 0:
                pltpu.sync_copy(src_ref.at[t, pl.ds(rbase, RPT)], idxs_v)
                x_ref = xs[slab2x[t]]
                for j in range(RPT):
                    pltpu.async_copy(x_ref.at[idxs_v.at[j]], rows_v, sem).wait()
                    pltpu.sync_copy(rows_v, accg.at[idxd_v.at[j]], add=True)
            else:
                for j in range(RPT):
                    pltpu.sync_copy(ones_v, accg.at[idxd_v.at[j]], add=True)
            plsc.subcore_barrier()
            pltpu.sync_copy(accg.at[my_rows], out_ref.at[t, c, my_rows])
            plsc.subcore_barrier()

    zeros = jnp.zeros((ROWS_PER_TILE, 128), jnp.float32)
    ones = jnp.ones((128, 128), jnp.float32)
    fn = pl.kernel(body, out_type=out_type, mesh=mesh, scratch_types=scratch)
    return fn(*x_slabs, src, dst, zeros, ones)


def _sc_gather(tables, idxs, dims):
    P = len(tables)
    mesh = plsc.VectorSubcoreMesh(core_axis_name="c", subcore_axis_name="s")
    Gs = [idx.shape[0] * 128 for idx in idxs]
    RPTs = [G // 128 // (NC * NS) for G in Gs]
    out_type = [jax.ShapeDtypeStruct((G, D), jnp.float32) for G, D in zip(Gs, dims)]
    scratch = []
    for i in range(P):
        scratch.append(pltpu.VMEM((RPTs[i], 128), jnp.int32))
        scratch.append(pltpu.VMEM((128, dims[i]), jnp.float32))
    scratch.append(pltpu.SemaphoreType.DMA)

    def body(*refs):
        tbls = refs[:P]
        idxr = refs[P:2 * P]
        outs = refs[2 * P:3 * P]
        sem = refs[-1]
        c = lax.axis_index("c")
        s = lax.axis_index("s")
        w = s * NC + c
        for i in range(P):
            idx_v, rows_v = refs[3 * P + 2 * i], refs[3 * P + 2 * i + 1]
            rbase = w * RPTs[i]
            pltpu.sync_copy(idxr[i].at[pl.ds(rbase, RPTs[i])], idx_v)
            for j in range(RPTs[i]):
                pltpu.async_copy(tbls[i].at[idx_v.at[j]], rows_v, sem).wait()
                pltpu.sync_copy(rows_v, outs[i].at[pl.ds((rbase + j) * 128, 128)])

    fn = pl.kernel(body, out_type=out_type, mesh=mesh, scratch_types=scratch)
    return fn(*tables, *idxs)


def _bigru_kernel_body(T_ref, xwf_ref, xwb_ref, wtf_ref, wtb_ref, bhf_ref, bhb_ref,
                       of_ref, ob_ref, hf_s, hb_s):
    c = pl.program_id(0)

    @pl.when(c == 0)
    def _():
        hf_s[...] = jnp.zeros_like(hf_s)
        hb_s[...] = jnp.zeros_like(hb_s)

    @pl.when(c * CH < T_ref[0])
    def _():
        wtf = wtf_ref[...]; wtb = wtb_ref[...]
        bhf = bhf_ref[...]; bhb = bhb_ref[...]

        def gate(xw, gh, h):
            r = jax.nn.sigmoid(xw[:, :GH] + gh[:, :GH])
            z = jax.nn.sigmoid(xw[:, GH:2 * GH] + gh[:, GH:2 * GH])
            nn_ = jnp.tanh(xw[:, 2 * GH:] + r * gh[:, 2 * GH:])
            return (1.0 - z) * nn_ + z * h

        def step(t, carry):
            hf, hb = carry
            ghf = jnp.dot(hf, wtf, preferred_element_type=jnp.float32) + bhf
            ghb = jnp.dot(hb, wtb, preferred_element_type=jnp.float32) + bhb
            h2f = gate(xwf_ref[t], ghf, hf)
            h2b = gate(xwb_ref[t], ghb, hb)
            of_ref[t] = h2f
            ob_ref[t] = h2b
            return (h2f, h2b)

        hf, hb = lax.fori_loop(0, CH, step, (hf_s[...], hb_s[...]))
        hf_s[...] = hf
        hb_s[...] = hb


def _bigru_pallas(xwf, xwb, WTf, WTb, bhf, bhb, tmax):
    grid_spec = pltpu.PrefetchScalarGridSpec(
        num_scalar_prefetch=1,
        grid=(LMAX // CH,),
        in_specs=[
            pl.BlockSpec((CH, B, 3 * GH), lambda i, T: (i, 0, 0)),
            pl.BlockSpec((CH, B, 3 * GH), lambda i, T: (i, 0, 0)),
            pl.BlockSpec((GH, 3 * GH), lambda i, T: (0, 0)),
            pl.BlockSpec((GH, 3 * GH), lambda i, T: (0, 0)),
            pl.BlockSpec((1, 3 * GH), lambda i, T: (0, 0)),
            pl.BlockSpec((1, 3 * GH), lambda i, T: (0, 0)),
        ],
        out_specs=[
            pl.BlockSpec((CH, B, GH), lambda i, T: (i, 0, 0)),
            pl.BlockSpec((CH, B, GH), lambda i, T: (i, 0, 0)),
        ],
        scratch_shapes=[
            pltpu.VMEM((B, GH), jnp.float32),
            pltpu.VMEM((B, GH), jnp.float32),
        ],
    )
    return pl.pallas_call(
        _bigru_kernel_body,
        grid_spec=grid_spec,
        out_shape=[
            jax.ShapeDtypeStruct((LMAX, B, GH), jnp.float32),
            jax.ShapeDtypeStruct((LMAX, B, GH), jnp.float32),
        ],
    )(tmax, xwf, xwb, WTf, WTb, bhf, bhb)


def _ln(v, w, b):
    m = v.mean(-1, keepdims=True)
    va = ((v - m) ** 2).mean(-1, keepdims=True)
    return (v - m) / jnp.sqrt(va + 1e-5) * w + b


def _mlp_pallas(v, W1, b1, lnw, lnb, W2, b2):
    dout = W2.shape[1]
    def body(v_ref, W1_ref, b1_ref, lnw_ref, lnb_ref, W2_ref, b2_ref, o_ref):
        u = jnp.maximum(jnp.dot(v_ref[...], W1_ref[...],
                                preferred_element_type=jnp.float32) + b1_ref[...], 0.0)
        m = u.mean(-1, keepdims=True)
        va = ((u - m) ** 2).mean(-1, keepdims=True)
        u = (u - m) / jnp.sqrt(va + 1e-5) * lnw_ref[...] + lnb_ref[...]
        o_ref[...] = jnp.dot(u, W2_ref[...], preferred_element_type=jnp.float32) + b2_ref[...]
    BLK = 1024
    return pl.pallas_call(
        body,
        grid=(N // BLK,),
        in_specs=[
            pl.BlockSpec((BLK, v.shape[1]), lambda i: (i, 0)),
            pl.BlockSpec(W1.shape, lambda i: (0, 0)),
            pl.BlockSpec((1, b1.shape[0]), lambda i: (0, 0)),
            pl.BlockSpec((1, lnw.shape[0]), lambda i: (0, 0)),
            pl.BlockSpec((1, lnb.shape[0]), lambda i: (0, 0)),
            pl.BlockSpec(W2.shape, lambda i: (0, 0)),
            pl.BlockSpec((1, b2.shape[0]), lambda i: (0, 0)),
        ],
        out_specs=pl.BlockSpec((BLK, dout), lambda i: (i, 0)),
        out_shape=jax.ShapeDtypeStruct((N, dout), jnp.float32),
    )(v, W1, b1[None], lnw[None], lnb[None], W2, b2[None])


def kernel(x_note, params, edge_index_onset, edge_index_consecutive, edge_index_during, neighbor_mask_note, batch):
    p = params
    ei_on, ei_co, ei_du = edge_index_onset, edge_index_consecutive, edge_index_during
    mask = neighbor_mask_note
    n = x_note.shape[0]

    src3 = jnp.stack([ei_on[0], ei_co[0], ei_du[0]]).reshape(3, E // 128, 128)
    dst3 = jnp.stack([ei_on[1], ei_co[1], ei_du[1]]).reshape(3, E // 128, 128)
    out1 = _sc_segsum([x_note], jnp.concatenate([src3, src3], 0),
                      jnp.concatenate([dst3, dst3], 0),
                      (0, 0, 0, -1, -1, -1))
    sums1 = out1[:3]
    cnt3 = jnp.clip(out1[3:, 0, :, 0] + out1[3:, 1, :, 0], 1.0)[..., None]
    mean1 = (sums1[:, 0] + sums1[:, 1]) / cnt3
    h = x_note @ p['l1_root_W'] + p['l1_root_b']
    for t, name in enumerate(('onset_W', 'consecutive_W', 'during_W')):
        h = h + mean1[t] @ p['l1_' + name]
    h = jax.nn.relu(h)

    src6 = jnp.stack([src3[0], src3[0], src3[1], src3[1], src3[2], src3[2]])
    dst6 = jnp.stack([dst3[0], dst3[0], dst3[1], dst3[1], dst3[2], dst3[2]])
    sums2 = _sc_segsum([h[:, :128], h[:, 128:]], src6, dst6,
                       (0, 1, 0, 1, 0, 1))
    sums2 = sums2[:, 0] + sums2[:, 1]
    h2 = h @ p['l2_root_W'] + p['l2_root_b']
    for t, name in enumerate(('onset_W', 'consecutive_W', 'during_W')):
        mean2 = jnp.concatenate([sums2[2 * t], sums2[2 * t + 1]], -1) / cnt3[t]
        h2 = h2 + mean2 @ p['l2_' + name]
    h = h2
    lengths = jnp.bincount(batch, length=B)
    cnt = jnp.clip(lengths, 1).astype(h.dtype)[:, None]
    mean = jax.ops.segment_sum(h, batch, num_segments=B) / cnt
    hc = h - p['gn_mean_scale'] * mean[batch]
    var = jax.ops.segment_sum(hc * hc, batch, num_segments=B) / cnt
    h = hc / jnp.sqrt(var[batch] + 1e-5) * p['gn_weight'] + p['gn_bias']

    starts = jnp.concatenate([jnp.zeros((1,), lengths.dtype), jnp.cumsum(lengths)[:-1]])
    pos = jnp.arange(n) - starts[batch]
    rev = lengths[batch] - 1 - pos
    tmax = jnp.max(lengths).astype(jnp.int32)[None]
    t_ar = jnp.arange(LMAX)[:, None]
    valid = t_ar < lengths[None, :]
    idx_f = jnp.where(valid, starts[None, :] + t_ar, n)
    idx_bk = jnp.where(valid, starts[None, :] + lengths[None, :] - 1 - t_ar, n)

    upf_idx = (pos * B + batch).astype(jnp.int32).reshape(-1, 128)
    upb_idx = (rev * B + batch).astype(jnp.int32).reshape(-1, 128)

    def bigru(flat, pre):
        xwf_flat = flat @ p[pre + 'f_Wih'].T + p[pre + 'f_bih']
        xwb_flat = flat @ p[pre + 'b_Wih'].T + p[pre + 'b_bih']
        extf = jnp.concatenate([xwf_flat, p[pre + 'f_bih'][None]], 0)
        extb = jnp.concatenate([xwb_flat, p[pre + 'b_bih'][None]], 0)
        xwf, xwb = _sc_gather(
            [extf, extb],
            [idx_f.reshape(-1, 128).astype(jnp.int32),
             idx_bk.reshape(-1, 128).astype(jnp.int32)],
            [3 * GH, 3 * GH])
        of, ob = _bigru_pallas(xwf.reshape(LMAX, B, 3 * GH),
                               xwb.reshape(LMAX, B, 3 * GH),
                               p[pre + 'f_Whh'].T, p[pre + 'b_Whh'].T,
                               p[pre + 'f_bhh'][None], p[pre + 'b_bhh'][None], tmax)
        zf, zb = _sc_gather(
            [of.reshape(LMAX * B, GH), ob.reshape(LMAX * B, GH)],
            [upf_idx, upb_idx], [GH, GH])
        return jnp.concatenate([zf, zb], axis=-1)

    znote = jnp.where((mask == 0)[:, None], x_note, jnp.zeros_like(x_note))
    z = bigru(znote, 'rnn_')
    z = _ln(z, p['ln1_w'], p['ln1_b'])
    z = z @ p['proj_W'] + p['proj_b']
    x2 = jnp.concatenate([h, z], axis=-1) @ p['cat_W'] + p['cat_b']
    out_pc = _mlp_pallas(x2, p['pc1_W'], p['pc1_b'], p['pc_ln_w'], p['pc_ln_b'], p['pc2_W'], p['pc2_b'])
    x3 = jnp.concatenate([x2, out_pc], axis=-1)
    x3 = bigru(x3, 'rnnks_')
    x3 = _ln(x3, p['lnks_w'], p['lnks_b'])
    x3 = x3 @ p['projks_W'] + p['projks_b']
    out_ks = _mlp_pallas(x3, p['ks1_W'], p['ks1_b'], p['ks_ln_w'], p['ks_ln_b'], p['ks2_W'], p['ks2_b'])
    return out_pc, out_ks


# speedup vs baseline: 8.5314x; 1.0130x over previous
import functools

import jax
import jax.numpy as jnp
from jax import lax
from jax.experimental import pallas as pl
from jax.experimental.pallas import tpu as pltpu
from jax.experimental.pallas import tpu_sc as plsc

N = 8192; B = 8; E = 65536; LMAX = 2048
IN = 128; H = 256; ENC = 256; PC = 35; KS = 15; GH = 128
CH = 128

NC, NS = 2, 16
ROWS_PER_TILE = N // NS


def _sc_segsum(x_slabs, src, dst, slab2x):
    S = src.shape[0]
    K = len(x_slabs)
    EPC = E // NC
    RPT = EPC // NS // 128
    mesh = plsc.VectorSubcoreMesh(core_axis_name="c", subcore_axis_name="s")

    out_type = jax.ShapeDtypeStruct((S, NC, N, 128), jnp.float32)

    NBUF = 2
    scratch = [
        pltpu.VMEM_SHARED((N, 128), jnp.float32),
        pltpu.VMEM((RPT, 128), jnp.int32),
        pltpu.VMEM((RPT, 128), jnp.int32),
        pltpu.VMEM((NBUF, 128, 128), jnp.float32),
        pltpu.VMEM((128, 128), jnp.float32),
        pltpu.SemaphoreType.DMA((NBUF,)),
        pltpu.SemaphoreType.DMA((NBUF,)),
    ]

    def body(*refs):
        xs = refs[:K]
        src_ref, dst_ref, zeros_ref, ones_ref, out_ref = refs[K:K + 5]
        accg, idxs_v, idxd_v, rows_v, ones_v, gsem, asem = refs[K + 5:]
        c = lax.axis_index("c")
        s = lax.axis_index("s")
        my_rows = pl.ds(s * ROWS_PER_TILE, ROWS_PER_TILE)
        rbase = c * (EPC // 128) + s * RPT
        if any(t < 0 for t in slab2x):
            pltpu.sync_copy(ones_ref, ones_v)
        for t in range(S):
            pltpu.sync_copy(zeros_ref, accg.at[my_rows])
            plsc.subcore_barrier()
            pltpu.sync_copy(dst_ref.at[t, pl.ds(rbase, RPT)], idxd_v)
            if slab2x[t] >= 0:
                pltpu.sync_copy(src_ref.at[t, pl.ds(rbase, RPT)], idxs_v)
                x_ref = xs[slab2x[t]]
                for g in range(RPT // NBUF):
                    gds = [pltpu.async_copy(x_ref.at[idxs_v.at[g * NBUF + k]],
                                            rows_v.at[k], gsem.at[k])
                           for k in range(NBUF)]
                    ads = []
                    for k in range(NBUF):
                        gds[k].wait()
                        ads.append(pltpu.async_copy(
                            rows_v.at[k], accg.at[idxd_v.at[g * NBUF + k]],
                            asem.at[k], add=True))
                    for d in ads:
                        d.wait()
            else:
                ads = [pltpu.async_copy(ones_v, accg.at[idxd_v.at[j]],
                                        asem.at[j % NBUF], add=True)
                       for j in range(RPT)]
                for d in ads:
                    d.wait()
            plsc.subcore_barrier()
            pltpu.sync_copy(accg.at[my_rows], out_ref.at[t, c, my_rows])
            plsc.subcore_barrier()

    zeros = jnp.zeros((ROWS_PER_TILE, 128), jnp.float32)
    ones = jnp.ones((128, 128), jnp.float32)
    fn = pl.kernel(body, out_type=out_type, mesh=mesh, scratch_types=scratch)
    return fn(*x_slabs, src, dst, zeros, ones)


def _sc_gather(tables, idxs, dims):
    P = len(tables)
    mesh = plsc.VectorSubcoreMesh(core_axis_name="c", subcore_axis_name="s")
    CHK = 64
    Gs = [idx.shape[0] * CHK for idx in idxs]
    RPTs = [G // CHK // (NC * NS) for G in Gs]
    out_type = [jax.ShapeDtypeStruct((G, D), jnp.float32) for G, D in zip(Gs, dims)]
    NBUF = 2
    Dmax = max(dims)
    assert all(d == Dmax for d in dims)
    scratch = [pltpu.VMEM((RPTs[i], CHK), jnp.int32) for i in range(P)]
    scratch.append(pltpu.VMEM((NBUF, CHK, Dmax), jnp.float32))
    scratch.append(pltpu.SemaphoreType.DMA((NBUF,)))
    scratch.append(pltpu.SemaphoreType.DMA((NBUF,)))

    def body(*refs):
        tbls = refs[:P]
        idxr = refs[P:2 * P]
        outs = refs[2 * P:3 * P]
        idx_vs = refs[3 * P:4 * P]
        rows_v, gsem, osem = refs[-3], refs[-2], refs[-1]
        c = lax.axis_index("c")
        s = lax.axis_index("s")
        w = s * NC + c
        for i in range(P):
            idx_v = idx_vs[i]
            rbase = w * RPTs[i]
            pltpu.sync_copy(idxr[i].at[pl.ds(rbase, RPTs[i])], idx_v)
            for g in range((RPTs[i] + NBUF - 1) // NBUF):
                ks = [k for k in range(NBUF) if g * NBUF + k < RPTs[i]]
                gds = [pltpu.async_copy(tbls[i].at[idx_v.at[g * NBUF + k]],
                                        rows_v.at[k], gsem.at[k])
                       for k in ks]
                ods = []
                for k in ks:
                    gds[k].wait()
                    j = g * NBUF + k
                    ods.append(pltpu.async_copy(
                        rows_v.at[k],
                        outs[i].at[pl.ds((rbase + j) * CHK, CHK)],
                        osem.at[k]))
                for d in ods:
                    d.wait()

    fn = pl.kernel(body, out_type=out_type, mesh=mesh, scratch_types=scratch)
    return fn(*tables, *idxs)


def _bigru_kernel_body(T_ref, xwf_ref, xwb_ref, wtf_ref, wtb_ref, bhf_ref, bhb_ref,
                       of_ref, ob_ref, hf_s, hb_s):
    c = pl.program_id(0)

    @pl.when(c == 0)
    def _():
        hf_s[...] = jnp.zeros_like(hf_s)
        hb_s[...] = jnp.zeros_like(hb_s)

    @pl.when(c * CH < T_ref[0])
    def _():
        wtf = wtf_ref[...]; wtb = wtb_ref[...]
        bhf = bhf_ref[...]; bhb = bhb_ref[...]

        def gate(xw, gh, h):
            r = jax.nn.sigmoid(xw[:, :GH] + gh[:, :GH])
            z = jax.nn.sigmoid(xw[:, GH:2 * GH] + gh[:, GH:2 * GH])
            nn_ = jnp.tanh(xw[:, 2 * GH:] + r * gh[:, 2 * GH:])
            return (1.0 - z) * nn_ + z * h

        def step(t, carry):
            hf, hb = carry
            ghf = jnp.dot(hf, wtf, preferred_element_type=jnp.float32) + bhf
            ghb = jnp.dot(hb, wtb, preferred_element_type=jnp.float32) + bhb
            h2f = gate(xwf_ref[t], ghf, hf)
            h2b = gate(xwb_ref[t], ghb, hb)
            of_ref[t] = h2f
            ob_ref[t] = h2b
            return (h2f, h2b)

        hf, hb = lax.fori_loop(0, CH, step, (hf_s[...], hb_s[...]))
        hf_s[...] = hf
        hb_s[...] = hb


def _bigru_pallas(xwf, xwb, WTf, WTb, bhf, bhb, tmax):
    grid_spec = pltpu.PrefetchScalarGridSpec(
        num_scalar_prefetch=1,
        grid=(LMAX // CH,),
        in_specs=[
            pl.BlockSpec((CH, B, 3 * GH), lambda i, T: (i, 0, 0)),
            pl.BlockSpec((CH, B, 3 * GH), lambda i, T: (i, 0, 0)),
            pl.BlockSpec((GH, 3 * GH), lambda i, T: (0, 0)),
            pl.BlockSpec((GH, 3 * GH), lambda i, T: (0, 0)),
            pl.BlockSpec((1, 3 * GH), lambda i, T: (0, 0)),
            pl.BlockSpec((1, 3 * GH), lambda i, T: (0, 0)),
        ],
        out_specs=[
            pl.BlockSpec((CH, B, GH), lambda i, T: (i, 0, 0)),
            pl.BlockSpec((CH, B, GH), lambda i, T: (i, 0, 0)),
        ],
        scratch_shapes=[
            pltpu.VMEM((B, GH), jnp.float32),
            pltpu.VMEM((B, GH), jnp.float32),
        ],
    )
    return pl.pallas_call(
        _bigru_kernel_body,
        grid_spec=grid_spec,
        out_shape=[
            jax.ShapeDtypeStruct((LMAX, B, GH), jnp.float32),
            jax.ShapeDtypeStruct((LMAX, B, GH), jnp.float32),
        ],
    )(tmax, xwf, xwb, WTf, WTb, bhf, bhb)


def _ln(v, w, b):
    m = v.mean(-1, keepdims=True)
    va = ((v - m) ** 2).mean(-1, keepdims=True)
    return (v - m) / jnp.sqrt(va + 1e-5) * w + b


def _mlp_pallas(v, W1, b1, lnw, lnb, W2, b2):
    dout = W2.shape[1]
    def body(v_ref, W1_ref, b1_ref, lnw_ref, lnb_ref, W2_ref, b2_ref, o_ref):
        u = jnp.maximum(jnp.dot(v_ref[...], W1_ref[...],
                                preferred_element_type=jnp.float32) + b1_ref[...], 0.0)
        m = u.mean(-1, keepdims=True)
        va = ((u - m) ** 2).mean(-1, keepdims=True)
        u = (u - m) / jnp.sqrt(va + 1e-5) * lnw_ref[...] + lnb_ref[...]
        o_ref[...] = jnp.dot(u, W2_ref[...], preferred_element_type=jnp.float32) + b2_ref[...]
    BLK = 1024
    return pl.pallas_call(
        body,
        grid=(N // BLK,),
        in_specs=[
            pl.BlockSpec((BLK, v.shape[1]), lambda i: (i, 0)),
            pl.BlockSpec(W1.shape, lambda i: (0, 0)),
            pl.BlockSpec((1, b1.shape[0]), lambda i: (0, 0)),
            pl.BlockSpec((1, lnw.shape[0]), lambda i: (0, 0)),
            pl.BlockSpec((1, lnb.shape[0]), lambda i: (0, 0)),
            pl.BlockSpec(W2.shape, lambda i: (0, 0)),
            pl.BlockSpec((1, b2.shape[0]), lambda i: (0, 0)),
        ],
        out_specs=pl.BlockSpec((BLK, dout), lambda i: (i, 0)),
        out_shape=jax.ShapeDtypeStruct((N, dout), jnp.float32),
    )(v, W1, b1[None], lnw[None], lnb[None], W2, b2[None])


def kernel(x_note, params, edge_index_onset, edge_index_consecutive, edge_index_during, neighbor_mask_note, batch):
    p = params
    ei_on, ei_co, ei_du = edge_index_onset, edge_index_consecutive, edge_index_during
    mask = neighbor_mask_note
    n = x_note.shape[0]

    src3 = jnp.stack([ei_on[0], ei_co[0], ei_du[0]]).reshape(3, E // 128, 128)
    dst3 = jnp.stack([ei_on[1], ei_co[1], ei_du[1]]).reshape(3, E // 128, 128)
    out1 = _sc_segsum([x_note], jnp.concatenate([src3, src3], 0),
                      jnp.concatenate([dst3, dst3], 0),
                      (0, 0, 0, -1, -1, -1))
    sums1 = out1[:3]
    cnt3 = jnp.clip(out1[3:, 0, :, 0] + out1[3:, 1, :, 0], 1.0)[..., None]
    mean1 = (sums1[:, 0] + sums1[:, 1]) / cnt3
    h = x_note @ p['l1_root_W'] + p['l1_root_b']
    for t, name in enumerate(('onset_W', 'consecutive_W', 'during_W')):
        h = h + mean1[t] @ p['l1_' + name]
    h = jax.nn.relu(h)

    src6 = jnp.stack([src3[0], src3[0], src3[1], src3[1], src3[2], src3[2]])
    dst6 = jnp.stack([dst3[0], dst3[0], dst3[1], dst3[1], dst3[2], dst3[2]])
    sums2 = _sc_segsum([h[:, :128], h[:, 128:]], src6, dst6,
                       (0, 1, 0, 1, 0, 1))
    sums2 = sums2[:, 0] + sums2[:, 1]
    h2 = h @ p['l2_root_W'] + p['l2_root_b']
    for t, name in enumerate(('onset_W', 'consecutive_W', 'during_W')):
        mean2 = jnp.concatenate([sums2[2 * t], sums2[2 * t + 1]], -1) / cnt3[t]
        h2 = h2 + mean2 @ p['l2_' + name]
    h = h2
    lengths = jnp.bincount(batch, length=B)
    cnt = jnp.clip(lengths, 1).astype(h.dtype)[:, None]
    mean = jax.ops.segment_sum(h, batch, num_segments=B) / cnt
    hc = h - p['gn_mean_scale'] * mean[batch]
    var = jax.ops.segment_sum(hc * hc, batch, num_segments=B) / cnt
    h = hc / jnp.sqrt(var[batch] + 1e-5) * p['gn_weight'] + p['gn_bias']

    starts = jnp.concatenate([jnp.zeros((1,), lengths.dtype), jnp.cumsum(lengths)[:-1]])
    pos = jnp.arange(n) - starts[batch]
    rev = lengths[batch] - 1 - pos
    tmax = jnp.max(lengths).astype(jnp.int32)[None]
    t_ar = jnp.arange(LMAX)[:, None]
    valid = t_ar < lengths[None, :]
    idx_f = jnp.where(valid, starts[None, :] + t_ar, n)
    idx_bk = jnp.where(valid, starts[None, :] + lengths[None, :] - 1 - t_ar, n)

    upf_idx = (pos * B + batch).astype(jnp.int32).reshape(-1, 64)
    upb_idx = (rev * B + batch).astype(jnp.int32).reshape(-1, 64)

    def bigru(flat, pre):
        xwf_flat = flat @ p[pre + 'f_Wih'].T + p[pre + 'f_bih']
        xwb_flat = flat @ p[pre + 'b_Wih'].T + p[pre + 'b_bih']
        extf = jnp.concatenate([xwf_flat, p[pre + 'f_bih'][None]], 0)
        extb = jnp.concatenate([xwb_flat, p[pre + 'b_bih'][None]], 0)
        xwf, xwb = _sc_gather(
            [extf, extb],
            [idx_f.reshape(-1, 64).astype(jnp.int32),
             idx_bk.reshape(-1, 64).astype(jnp.int32)],
            [3 * GH, 3 * GH])
        of, ob = _bigru_pallas(xwf.reshape(LMAX, B, 3 * GH),
                               xwb.reshape(LMAX, B, 3 * GH),
                               p[pre + 'f_Whh'].T, p[pre + 'b_Whh'].T,
                               p[pre + 'f_bhh'][None], p[pre + 'b_bhh'][None], tmax)
        zf, zb = _sc_gather(
            [of.reshape(LMAX * B, GH), ob.reshape(LMAX * B, GH)],
            [upf_idx, upb_idx], [GH, GH])
        return jnp.concatenate([zf, zb], axis=-1)

    znote = jnp.where((mask == 0)[:, None], x_note, jnp.zeros_like(x_note))
    z = bigru(znote, 'rnn_')
    z = _ln(z, p['ln1_w'], p['ln1_b'])
    z = z @ p['proj_W'] + p['proj_b']
    x2 = jnp.concatenate([h, z], axis=-1) @ p['cat_W'] + p['cat_b']
    out_pc = _mlp_pallas(x2, p['pc1_W'], p['pc1_b'], p['pc_ln_w'], p['pc_ln_b'], p['pc2_W'], p['pc2_b'])
    x3 = jnp.concatenate([x2, out_pc], axis=-1)
    x3 = bigru(x3, 'rnnks_')
    x3 = _ln(x3, p['lnks_w'], p['lnks_b'])
    x3 = x3 @ p['projks_W'] + p['projks_b']
    out_ks = _mlp_pallas(x3, p['ks1_W'], p['ks1_b'], p['ks_ln_w'], p['ks_ln_b'], p['ks2_W'], p['ks2_b'])
    return out_pc, out_ks


# speedup vs baseline: 9.9562x; 1.1670x over previous
import functools

import jax
import jax.numpy as jnp
from jax import lax
from jax.experimental import pallas as pl
from jax.experimental.pallas import tpu as pltpu
from jax.experimental.pallas import tpu_sc as plsc

N = 8192; B = 8; E = 65536; LMAX = 2048
IN = 128; H = 256; ENC = 256; PC = 35; KS = 15; GH = 128
CH = 128

NC, NS = 2, 16
ROWS_PER_TILE = N // NS


def _sc_segsum(x_slabs, src, dst, slab2x):
    S = src.shape[0]
    K = len(x_slabs)
    EPC = E // NC
    RPT = EPC // NS // 128
    mesh = plsc.VectorSubcoreMesh(core_axis_name="c", subcore_axis_name="s")

    out_type = jax.ShapeDtypeStruct((S, NC, N, 128), jnp.float32)

    NBUF = 2
    scratch = [
        pltpu.VMEM_SHARED((N, 128), jnp.float32),
        pltpu.VMEM((RPT, 128), jnp.int32),
        pltpu.VMEM((RPT, 128), jnp.int32),
        pltpu.VMEM((NBUF, 128, 128), jnp.float32),
        pltpu.VMEM((128, 128), jnp.float32),
        pltpu.SemaphoreType.DMA((NBUF,)),
        pltpu.SemaphoreType.DMA((NBUF,)),
    ]

    def body(*refs):
        xs = refs[:K]
        src_ref, dst_ref, zeros_ref, ones_ref, out_ref = refs[K:K + 5]
        accg, idxs_v, idxd_v, rows_v, ones_v, gsem, asem = refs[K + 5:]
        c = lax.axis_index("c")
        s = lax.axis_index("s")
        my_rows = pl.ds(s * ROWS_PER_TILE, ROWS_PER_TILE)
        rbase = c * (EPC // 128) + s * RPT
        if any(t < 0 for t in slab2x):
            pltpu.sync_copy(ones_ref, ones_v)
        for t in range(S):
            pltpu.sync_copy(zeros_ref, accg.at[my_rows])
            plsc.subcore_barrier()
            pltpu.sync_copy(dst_ref.at[t, pl.ds(rbase, RPT)], idxd_v)
            if slab2x[t] >= 0:
                pltpu.sync_copy(src_ref.at[t, pl.ds(rbase, RPT)], idxs_v)
                x_ref = xs[slab2x[t]]
                for g in range(RPT // NBUF):
                    gds = [pltpu.async_copy(x_ref.at[idxs_v.at[g * NBUF + k]],
                                            rows_v.at[k], gsem.at[k])
                           for k in range(NBUF)]
                    ads = []
                    for k in range(NBUF):
                        gds[k].wait()
                        ads.append(pltpu.async_copy(
                            rows_v.at[k], accg.at[idxd_v.at[g * NBUF + k]],
                            asem.at[k], add=True))
                    for d in ads:
                        d.wait()
            else:
                ads = [pltpu.async_copy(ones_v, accg.at[idxd_v.at[j]],
                                        asem.at[j % NBUF], add=True)
                       for j in range(RPT)]
                for d in ads:
                    d.wait()
            plsc.subcore_barrier()
            pltpu.sync_copy(accg.at[my_rows], out_ref.at[t, c, my_rows])
            plsc.subcore_barrier()

    zeros = jnp.zeros((ROWS_PER_TILE, 128), jnp.float32)
    ones = jnp.ones((128, 128), jnp.float32)
    fn = pl.kernel(body, out_type=out_type, mesh=mesh, scratch_types=scratch)
    return fn(*x_slabs, src, dst, zeros, ones)


def _sc_gather(tables, idxs, dims):
    P = len(tables)
    mesh = plsc.VectorSubcoreMesh(core_axis_name="c", subcore_axis_name="s")
    CHK = 64
    Gs = [idx.shape[0] * CHK for idx in idxs]
    RPTs = [G // CHK // (NC * NS) for G in Gs]
    out_type = [jax.ShapeDtypeStruct((G, D), jnp.float32) for G, D in zip(Gs, dims)]
    NBUF = 2
    Dmax = max(dims)
    assert all(d == Dmax for d in dims)
    scratch = [pltpu.VMEM((RPTs[i], CHK), jnp.int32) for i in range(P)]
    scratch.append(pltpu.VMEM((NBUF, CHK, Dmax), jnp.float32))
    scratch.append(pltpu.SemaphoreType.DMA((NBUF,)))
    scratch.append(pltpu.SemaphoreType.DMA((NBUF,)))

    def body(*refs):
        tbls = refs[:P]
        idxr = refs[P:2 * P]
        outs = refs[2 * P:3 * P]
        idx_vs = refs[3 * P:4 * P]
        rows_v, gsem, osem = refs[-3], refs[-2], refs[-1]
        c = lax.axis_index("c")
        s = lax.axis_index("s")
        w = s * NC + c
        for i in range(P):
            idx_v = idx_vs[i]
            rbase = w * RPTs[i]
            pltpu.sync_copy(idxr[i].at[pl.ds(rbase, RPTs[i])], idx_v)
            for g in range((RPTs[i] + NBUF - 1) // NBUF):
                ks = [k for k in range(NBUF) if g * NBUF + k < RPTs[i]]
                gds = [pltpu.async_copy(tbls[i].at[idx_v.at[g * NBUF + k]],
                                        rows_v.at[k], gsem.at[k])
                       for k in ks]
                ods = []
                for k in ks:
                    gds[k].wait()
                    j = g * NBUF + k
                    ods.append(pltpu.async_copy(
                        rows_v.at[k],
                        outs[i].at[pl.ds((rbase + j) * CHK, CHK)],
                        osem.at[k]))
                for d in ods:
                    d.wait()

    fn = pl.kernel(body, out_type=out_type, mesh=mesh, scratch_types=scratch)
    return fn(*tables, *idxs)


def _bigru_kernel_body(T_ref, xwf_ref, xwb_ref, wtf_ref, wtb_ref, bhf_ref, bhb_ref,
                       of_ref, ob_ref, hf_s, hb_s):
    c = pl.program_id(0)

    @pl.when(c == 0)
    def _():
        hf_s[...] = jnp.zeros_like(hf_s)
        hb_s[...] = jnp.zeros_like(hb_s)

    @pl.when(c * CH < T_ref[0])
    def _():
        wtf = wtf_ref[...]; wtb = wtb_ref[...]
        bhf = bhf_ref[...]; bhb = bhb_ref[...]

        def gate(xw, gh, h):
            r = jax.nn.sigmoid(xw[:, :GH] + gh[:, :GH])
            z = jax.nn.sigmoid(xw[:, GH:2 * GH] + gh[:, GH:2 * GH])
            nn_ = jnp.tanh(xw[:, 2 * GH:] + r * gh[:, 2 * GH:])
            return (1.0 - z) * nn_ + z * h

        def step(t, carry):
            hf, hb = carry
            ghf = jnp.dot(hf, wtf, preferred_element_type=jnp.float32) + bhf
            ghb = jnp.dot(hb, wtb, preferred_element_type=jnp.float32) + bhb
            h2f = gate(xwf_ref[t], ghf, hf)
            h2b = gate(xwb_ref[t], ghb, hb)
            of_ref[t] = h2f
            ob_ref[t] = h2b
            return (h2f, h2b)

        hf, hb = lax.fori_loop(0, CH, step, (hf_s[...], hb_s[...]))
        hf_s[...] = hf
        hb_s[...] = hb


def _bigru_pallas(xwf, xwb, WTf, WTb, bhf, bhb, tmax):
    grid_spec = pltpu.PrefetchScalarGridSpec(
        num_scalar_prefetch=1,
        grid=(LMAX // CH,),
        in_specs=[
            pl.BlockSpec((CH, B, 3 * GH), lambda i, T: (i, 0, 0)),
            pl.BlockSpec((CH, B, 3 * GH), lambda i, T: (i, 0, 0)),
            pl.BlockSpec((GH, 3 * GH), lambda i, T: (0, 0)),
            pl.BlockSpec((GH, 3 * GH), lambda i, T: (0, 0)),
            pl.BlockSpec((1, 3 * GH), lambda i, T: (0, 0)),
            pl.BlockSpec((1, 3 * GH), lambda i, T: (0, 0)),
        ],
        out_specs=[
            pl.BlockSpec((CH, B, GH), lambda i, T: (i, 0, 0)),
            pl.BlockSpec((CH, B, GH), lambda i, T: (i, 0, 0)),
        ],
        scratch_shapes=[
            pltpu.VMEM((B, GH), jnp.float32),
            pltpu.VMEM((B, GH), jnp.float32),
        ],
    )
    return pl.pallas_call(
        _bigru_kernel_body,
        grid_spec=grid_spec,
        out_shape=[
            jax.ShapeDtypeStruct((LMAX, B, GH), jnp.float32),
            jax.ShapeDtypeStruct((LMAX, B, GH), jnp.float32),
        ],
    )(tmax, xwf, xwb, WTf, WTb, bhf, bhb)


def _mm_pallas(xs, Ws, b, relu=False, ln=None):
    BLK = 1024
    dout = Ws[0].shape[1]
    P = len(xs)

    def body(*refs):
        xrefs = refs[:P]
        wrefs = refs[P:2 * P]
        b_ref = refs[2 * P]
        if ln is not None:
            lnw_ref, lnb_ref = refs[2 * P + 1:2 * P + 3]
        o_ref = refs[-1]
        acc = None
        for i in range(P):
            v = xrefs[i][...]
            if ln is not None:
                m = v.mean(-1, keepdims=True)
                va = ((v - m) ** 2).mean(-1, keepdims=True)
                v = (v - m) / jnp.sqrt(va + 1e-5) * lnw_ref[...] + lnb_ref[...]
            d = jnp.dot(v, wrefs[i][...], preferred_element_type=jnp.float32)
            acc = d if acc is None else acc + d
        acc = acc + b_ref[...]
        o_ref[...] = jnp.maximum(acc, 0.0) if relu else acc

    in_specs = [pl.BlockSpec((BLK, x.shape[1]), lambda i: (i, 0)) for x in xs]
    in_specs += [pl.BlockSpec(W.shape, lambda i: (0, 0)) for W in Ws]
    in_specs += [pl.BlockSpec((1, dout), lambda i: (0, 0))]
    args = list(xs) + list(Ws) + [b[None]]
    if ln is not None:
        in_specs += [pl.BlockSpec((1, ln[0].shape[0]), lambda i: (0, 0))] * 2
        args += [ln[0][None], ln[1][None]]
    return pl.pallas_call(
        body,
        grid=(N // BLK,),
        in_specs=in_specs,
        out_specs=pl.BlockSpec((BLK, dout), lambda i: (i, 0)),
        out_shape=jax.ShapeDtypeStruct((N, dout), jnp.float32),
    )(*args)


def _hetero_pallas(x, sums, cnts, Wcat, b, relu, split):
    BLK = 1024
    din = x.shape[1]
    S = sums.shape[0]

    def body(x_ref, s_ref, c_ref, w_ref, b_ref, o_ref):
        acc = jnp.dot(x_ref[...], w_ref[pl.ds(0, din)],
                      preferred_element_type=jnp.float32) + b_ref[...]
        for t in range(3):
            rcp = 1.0 / jnp.maximum(c_ref[t, 0] + c_ref[t, 1], 1.0)
            if split:
                m0 = (s_ref[2 * t, 0] + s_ref[2 * t, 1]) * rcp
                m1 = (s_ref[2 * t + 1, 0] + s_ref[2 * t + 1, 1]) * rcp
                mean = jnp.concatenate([m0, m1], axis=-1)
            else:
                mean = (s_ref[t, 0] + s_ref[t, 1]) * rcp
            acc = acc + jnp.dot(mean, w_ref[pl.ds(din * (t + 1), din)],
                                preferred_element_type=jnp.float32)
        o_ref[...] = jnp.maximum(acc, 0.0) if relu else acc

    dout = Wcat.shape[1]
    return pl.pallas_call(
        body,
        grid=(N // BLK,),
        in_specs=[
            pl.BlockSpec((BLK, din), lambda i: (i, 0)),
            pl.BlockSpec((S, 2, BLK, 128), lambda i: (0, 0, i, 0)),
            pl.BlockSpec((3, 2, BLK, 128), lambda i: (0, 0, i, 0)),
            pl.BlockSpec(Wcat.shape, lambda i: (0, 0)),
            pl.BlockSpec((1, dout), lambda i: (0, 0)),
        ],
        out_specs=pl.BlockSpec((BLK, dout), lambda i: (i, 0)),
        out_shape=jax.ShapeDtypeStruct((N, dout), jnp.float32),
    )(x, sums, cnts, Wcat, b[None])


def _gn_pallas(h, M, MT, lengths_f, w, b, scale):
    def body(h_ref, m_ref, mt_ref, len_ref, w_ref, b_ref, sc_ref, o_ref):
        hi = lax.Precision.HIGHEST
        hv = h_ref[...]
        rcp = 1.0 / jnp.maximum(len_ref[...], 1.0)
        mean = jnp.dot(m_ref[...], hv, preferred_element_type=jnp.float32,
                       precision=hi) * rcp
        hc = hv - sc_ref[...] * jnp.dot(mt_ref[...], mean,
                                        preferred_element_type=jnp.float32,
                                        precision=hi)
        var = jnp.dot(m_ref[...], hc * hc, preferred_element_type=jnp.float32,
                      precision=hi) * rcp
        varb = jnp.dot(mt_ref[...], var, preferred_element_type=jnp.float32,
                       precision=hi)
        o_ref[...] = hc / jnp.sqrt(varb + 1e-5) * w_ref[...] + b_ref[...]

    CB = 128
    return pl.pallas_call(
        body,
        grid=(ENC // CB,),
        in_specs=[
            pl.BlockSpec((N, CB), lambda i: (0, i)),
            pl.BlockSpec((B, N), lambda i: (0, 0)),
            pl.BlockSpec((N, B), lambda i: (0, 0)),
            pl.BlockSpec((B, 1), lambda i: (0, 0)),
            pl.BlockSpec((1, CB), lambda i: (0, i)),
            pl.BlockSpec((1, CB), lambda i: (0, i)),
            pl.BlockSpec((1, CB), lambda i: (0, i)),
        ],
        out_specs=pl.BlockSpec((N, CB), lambda i: (0, i)),
        out_shape=jax.ShapeDtypeStruct((N, ENC), jnp.float32),
    )(h, M, MT, lengths_f, w[None], b[None], scale[None])


def _mlp_pallas(v, W1, b1, lnw, lnb, W2, b2):
    dout = W2.shape[1]
    def body(v_ref, W1_ref, b1_ref, lnw_ref, lnb_ref, W2_ref, b2_ref, o_ref):
        u = jnp.maximum(jnp.dot(v_ref[...], W1_ref[...],
                                preferred_element_type=jnp.float32) + b1_ref[...], 0.0)
        m = u.mean(-1, keepdims=True)
        va = ((u - m) ** 2).mean(-1, keepdims=True)
        u = (u - m) / jnp.sqrt(va + 1e-5) * lnw_ref[...] + lnb_ref[...]
        o_ref[...] = jnp.dot(u, W2_ref[...], preferred_element_type=jnp.float32) + b2_ref[...]
    BLK = 1024
    return pl.pallas_call(
        body,
        grid=(N // BLK,),
        in_specs=[
            pl.BlockSpec((BLK, v.shape[1]), lambda i: (i, 0)),
            pl.BlockSpec(W1.shape, lambda i: (0, 0)),
            pl.BlockSpec((1, b1.shape[0]), lambda i: (0, 0)),
            pl.BlockSpec((1, lnw.shape[0]), lambda i: (0, 0)),
            pl.BlockSpec((1, lnb.shape[0]), lambda i: (0, 0)),
            pl.BlockSpec(W2.shape, lambda i: (0, 0)),
            pl.BlockSpec((1, b2.shape[0]), lambda i: (0, 0)),
        ],
        out_specs=pl.BlockSpec((BLK, dout), lambda i: (i, 0)),
        out_shape=jax.ShapeDtypeStruct((N, dout), jnp.float32),
    )(v, W1, b1[None], lnw[None], lnb[None], W2, b2[None])


def kernel(x_note, params, edge_index_onset, edge_index_consecutive, edge_index_during, neighbor_mask_note, batch):
    p = params
    ei_on, ei_co, ei_du = edge_index_onset, edge_index_consecutive, edge_index_during
    mask = neighbor_mask_note
    n = x_note.shape[0]

    src3 = jnp.stack([ei_on[0], ei_co[0], ei_du[0]]).reshape(3, E // 128, 128)
    dst3 = jnp.stack([ei_on[1], ei_co[1], ei_du[1]]).reshape(3, E // 128, 128)
    out1 = _sc_segsum([x_note], jnp.concatenate([src3, src3], 0),
                      jnp.concatenate([dst3, dst3], 0),
                      (0, 0, 0, -1, -1, -1))
    cnts = out1[3:]
    W1 = jnp.concatenate([p['l1_root_W'], p['l1_onset_W'],
                          p['l1_consecutive_W'], p['l1_during_W']], 0)
    h = _hetero_pallas(x_note, out1[:3], cnts, W1, p['l1_root_b'], True, False)

    src6 = jnp.stack([src3[0], src3[0], src3[1], src3[1], src3[2], src3[2]])
    dst6 = jnp.stack([dst3[0], dst3[0], dst3[1], dst3[1], dst3[2], dst3[2]])
    sums2 = _sc_segsum([h[:, :128], h[:, 128:]], src6, dst6, (0, 1, 0, 1, 0, 1))
    W2 = jnp.concatenate([p['l2_root_W'], p['l2_onset_W'],
                          p['l2_consecutive_W'], p['l2_during_W']], 0)
    h = _hetero_pallas(h, sums2, cnts, W2, p['l2_root_b'], False, True)

    gidx = jnp.arange(B, dtype=batch.dtype)
    starts = jnp.searchsorted(batch, gidx).astype(jnp.int32)
    ends = jnp.searchsorted(batch, gidx, side='right').astype(jnp.int32)
    lengths = ends - starts
    Moh = (batch[None, :] == gidx[:, None]).astype(jnp.float32)
    h = _gn_pallas(h, Moh, Moh.T, lengths.astype(jnp.float32)[:, None],
                   p['gn_weight'], p['gn_bias'], p['gn_mean_scale'])

    pos = (jnp.arange(n, dtype=jnp.int32) - starts[batch])
    rev = lengths[batch] - 1 - pos
    tmax = jnp.max(lengths)[None]
    t_ar = jnp.arange(LMAX, dtype=jnp.int32)[:, None]
    valid = t_ar < lengths[None, :]
    idx_f = jnp.where(valid, starts[None, :] + t_ar, 0)
    idx_bk = jnp.where(valid, starts[None, :] + lengths[None, :] - 1 - t_ar, 0)

    upf_idx = (pos * B + batch).astype(jnp.int32).reshape(-1, 64)
    upb_idx = (rev * B + batch).astype(jnp.int32).reshape(-1, 64)

    def bigru(parts, dims, pre):
        def xw(d):
            WT = p[pre + d + 'Wih'].T
            off = 0
            Wsplit = []
            for dd in dims:
                Wsplit.append(WT[off:off + dd])
                off += dd
            return _mm_pallas(parts, Wsplit, p[pre + d + 'bih'])
        xwf_flat, xwb_flat = xw('f_'), xw('b_')
        xwf, xwb = _sc_gather(
            [xwf_flat, xwb_flat],
            [idx_f.reshape(-1, 64).astype(jnp.int32),
             idx_bk.reshape(-1, 64).astype(jnp.int32)],
            [3 * GH, 3 * GH])
        of, ob = _bigru_pallas(xwf.reshape(LMAX, B, 3 * GH),
                               xwb.reshape(LMAX, B, 3 * GH),
                               p[pre + 'f_Whh'].T, p[pre + 'b_Whh'].T,
                               p[pre + 'f_bhh'][None], p[pre + 'b_bhh'][None], tmax)
        zf, zb = _sc_gather(
            [of.reshape(LMAX * B, GH), ob.reshape(LMAX * B, GH)],
            [upf_idx, upb_idx], [GH, GH])
        return jnp.concatenate([zf, zb], axis=-1)

    znote = jnp.where((mask == 0)[:, None], x_note, jnp.zeros_like(x_note))
    z = bigru([znote], [IN], 'rnn_')
    z = _mm_pallas([z], [p['proj_W']], p['proj_b'], ln=(p['ln1_w'], p['ln1_b']))
    x2 = _mm_pallas([h, z], [p['cat_W'][:ENC], p['cat_W'][ENC:]], p['cat_b'])
    out_pc = _mlp_pallas(x2, p['pc1_W'], p['pc1_b'], p['pc_ln_w'], p['pc_ln_b'], p['pc2_W'], p['pc2_b'])
    x3 = bigru([x2, out_pc], [ENC, PC], 'rnnks_')
    x3 = _mm_pallas([x3], [p['projks_W']], p['projks_b'],
                    ln=(p['lnks_w'], p['lnks_b']))
    out_ks = _mlp_pallas(x3, p['ks1_W'], p['ks1_b'], p['ks_ln_w'], p['ks_ln_b'], p['ks2_W'], p['ks2_b'])
    return out_pc, out_ks
